# baseline (speedup 1.0000x reference)
import jax
import jax.numpy as jnp
from jax import lax
from jax.experimental import pallas as pl
from jax.experimental.pallas import tpu as pltpu
from jax.experimental.pallas import tpu_sc as plsc

F = 26
B = 4096
K = 64
H1 = 256
H2 = 128
V = 100000
NC = 2
NS = 16
NW = NC * NS
BPW = B // NW
CH = 32
NCH = BPW // CH
L = 16
KV = K // L


def _sc_a_body(x_hbm, emb_hbm, xbi_hbm, idx_v, rows_a, rows_b, xbi_stage,
               sem_a, sem_b):
    wid = lax.axis_index("s") * NC + lax.axis_index("c")
    base = wid * BPW

    for f in range(F):
        pltpu.sync_copy(x_hbm.at[pl.ds(f * B + base, BPW)], idx_v.at[f])

    bufs = (rows_a, rows_b)
    sems = (sem_a, sem_b)

    def fire(c):
        buf = bufs[c % 2]
        sem = sems[c % 2]
        hs = []
        for f in range(F):
            hs.append(pltpu.async_copy(
                emb_hbm.at[idx_v.at[f, pl.ds(c * CH, CH)]], buf.at[f], sem))
        return hs

    pending = fire(0)

    for c in range(NCH):
        nxt = fire(c + 1) if c + 1 < NCH else None
        for h_ in pending:
            h_.wait()
        pending = nxt
        buf = bufs[c % 2]

        def body(b, carry, buf=buf):
            v = [buf[0, b, pl.ds(k * L, L)] for k in range(KV)]
            s = list(v)
            q = [vk * vk for vk in v]
            for f in range(1, F):
                v = [buf[f, b, pl.ds(k * L, L)] for k in range(KV)]
                for k in range(KV):
                    s[k] = s[k] + v[k]
                    q[k] = q[k] + v[k] * v[k]
            for k in range(KV):
                xbi_stage[b, pl.ds(k * L, L)] = (s[k] * s[k] - q[k]) * 0.5
            return carry

        lax.fori_loop(0, CH, body, 0, unroll=False)
        pltpu.sync_copy(xbi_stage,
                        xbi_hbm.at[pl.ds(base + c * CH, CH), pl.ds(0, K)])


_sc_a_call = pl.kernel(
    _sc_a_body,
    out_type=[jax.ShapeDtypeStruct((B, 128), jnp.float32)],
    mesh=plsc.VectorSubcoreMesh(core_axis_name="c", subcore_axis_name="s"),
    scratch_types=[
        pltpu.VMEM((F, BPW), jnp.int32),
        pltpu.VMEM((F, CH, K), jnp.float32),
        pltpu.VMEM((F, CH, K), jnp.float32),
        pltpu.VMEM((CH, K), jnp.float32),
        pltpu.SemaphoreType.DMA,
        pltpu.SemaphoreType.DMA,
    ],
    compiler_params=pltpu.CompilerParams(use_tc_tiling_on_sc=False),
)


def _sc_b_body(x_hbm, w1_hbm, fm1_hbm, idx_v, w1_rows, fm1_stage, w1sem):
    wid = lax.axis_index("s") * NC + lax.axis_index("c")
    base = wid * BPW

    for f in range(F):
        pltpu.sync_copy(x_hbm.at[pl.ds(f * B + base, BPW)], idx_v.at[f])

    w1_pend = []
    for f in range(F):
        w1_pend.append(pltpu.async_copy(
            w1_hbm.at[idx_v.at[f]], w1_rows.at[f], w1sem))
    for h_ in w1_pend:
        h_.wait()

    for ch in range(BPW // L):
        acc = w1_rows[0, pl.ds(ch * L, L)]
        for f in range(1, F):
            acc = acc + w1_rows[f, pl.ds(ch * L, L)]
        fm1_stage[pl.ds(ch * L, L)] = acc
    pltpu.sync_copy(fm1_stage, fm1_hbm.at[pl.ds(base, BPW)])


_sc_b_call = pl.kernel(
    _sc_b_body,
    out_type=[jax.ShapeDtypeStruct((B,), jnp.float32)],
    mesh=plsc.VectorSubcoreMesh(core_axis_name="c", subcore_axis_name="s"),
    scratch_types=[
        pltpu.VMEM((F, BPW), jnp.int32),
        pltpu.VMEM((F, BPW), jnp.float32),
        pltpu.VMEM((BPW,), jnp.float32),
        pltpu.SemaphoreType.DMA,
    ],
    compiler_params=pltpu.CompilerParams(use_tc_tiling_on_sc=False),
)


def _tc_body(xbi_ref, fm1_ref, w0_ref, w1m_ref, b1_ref, w2m_ref, b2_ref,
             h_ref, out_ref):
    xbi = xbi_ref[:, :K]
    a1 = jnp.dot(xbi, w1m_ref[...], preferred_element_type=jnp.float32)
    a1 = jnp.maximum(a1 + b1_ref[...], 0.0)
    a2 = jnp.dot(a1, w2m_ref[...], preferred_element_type=jnp.float32)
    a2 = jnp.maximum(a2 + b2_ref[...], 0.0)
    out = jnp.dot(a2, h_ref[...], preferred_element_type=jnp.float32)
    out_ref[...] = out + fm1_ref[...] + w0_ref[0]


_TC_BLK = 512


def _tc_call(xbi, fm1_2d, w0_1, w1m, b1, w2m, b2, h):
    grid = (B // _TC_BLK,)
    return pl.pallas_call(
        _tc_body,
        grid=grid,
        in_specs=[
            pl.BlockSpec((_TC_BLK, 128), lambda i: (i, 0)),
            pl.BlockSpec((_TC_BLK, 1), lambda i: (i, 0)),
            pl.BlockSpec(memory_space=pltpu.SMEM),
            pl.BlockSpec((K, H1), lambda i: (0, 0)),
            pl.BlockSpec((H1,), lambda i: (0,)),
            pl.BlockSpec((H1, H2), lambda i: (0, 0)),
            pl.BlockSpec((H2,), lambda i: (0,)),
            pl.BlockSpec((H2, 1), lambda i: (0, 0)),
        ],
        out_specs=pl.BlockSpec((_TC_BLK, 1), lambda i: (i, 0)),
        out_shape=jax.ShapeDtypeStruct((B, 1), jnp.float32),
    )(xbi, fm1_2d, w0_1, w1m, b1, w2m, b2, h)


def kernel(x, emb_v, w1_table, w0, W1, b1, W2, b2, h):
    x1 = x.reshape(F * B)
    (xbi,) = _sc_a_call(x1, emb_v)
    (fm1,) = _sc_b_call(x1, w1_table.reshape(V))
    return _tc_call(xbi, fm1.reshape(B, 1), w0.reshape(1), W1, b1, W2, b2, h)


# speedup vs baseline: 1.0963x; 1.0963x over previous
import jax
import jax.numpy as jnp
from jax import lax
from jax.experimental import pallas as pl
from jax.experimental.pallas import tpu as pltpu
from jax.experimental.pallas import tpu_sc as plsc

F = 26
B = 4096
K = 64
H1 = 256
H2 = 128
V = 100000
NC = 2
NS = 16
NW = NC * NS
BPW = B // NW
CH = 32
NCH = BPW // CH
L = 16
KV = K // L


def _sc_a_body(x_hbm, emb_hbm, xbi_hbm, idx_v, rows_a, rows_b, xbi_stage,
               sem_a, sem_b):
    wid = lax.axis_index("s") * NC + lax.axis_index("c")
    base = wid * BPW

    pltpu.sync_copy(x_hbm.at[:, pl.ds(base, BPW)], idx_v)

    bufs = (rows_a, rows_b)
    sems = (sem_a, sem_b)

    def fire(c):
        buf = bufs[c % 2]
        sem = sems[c % 2]
        hs = []
        for f in range(F):
            hs.append(pltpu.async_copy(
                emb_hbm.at[idx_v.at[f, pl.ds(c * CH, CH)]], buf.at[f], sem))
        return hs

    pending = fire(0)

    for c in range(NCH):
        nxt = fire(c + 1) if c + 1 < NCH else None
        for h_ in pending:
            h_.wait()
        pending = nxt
        buf = bufs[c % 2]

        def body(b, carry, buf=buf):
            v = [buf[0, b, pl.ds(k * L, L)] for k in range(KV)]
            s = list(v)
            q = [vk * vk for vk in v]
            for f in range(1, F):
                v = [buf[f, b, pl.ds(k * L, L)] for k in range(KV)]
                for k in range(KV):
                    s[k] = s[k] + v[k]
                    q[k] = q[k] + v[k] * v[k]
            for k in range(KV):
                xbi_stage[b, pl.ds(k * L, L)] = (s[k] * s[k] - q[k]) * 0.5
            return carry

        lax.fori_loop(0, CH, body, 0, unroll=False)
        pltpu.sync_copy(xbi_stage,
                        xbi_hbm.at[pl.ds(base + c * CH, CH), pl.ds(0, K)])


_sc_a_call = pl.kernel(
    _sc_a_body,
    out_type=[jax.ShapeDtypeStruct((B, 128), jnp.float32)],
    mesh=plsc.VectorSubcoreMesh(core_axis_name="c", subcore_axis_name="s"),
    scratch_types=[
        pltpu.VMEM((F, BPW), jnp.int32),
        pltpu.VMEM((F, CH, K), jnp.float32),
        pltpu.VMEM((F, CH, K), jnp.float32),
        pltpu.VMEM((CH, K), jnp.float32),
        pltpu.SemaphoreType.DMA,
        pltpu.SemaphoreType.DMA,
    ],
    compiler_params=pltpu.CompilerParams(use_tc_tiling_on_sc=False),
)


def _sc_b_body(x_hbm, w1_hbm, fm1_hbm, idx_v, w1_rows, fm1_stage, w1sem):
    wid = lax.axis_index("s") * NC + lax.axis_index("c")
    base = wid * BPW

    pltpu.sync_copy(x_hbm.at[:, pl.ds(base, BPW)], idx_v)

    w1_pend = []
    for f in range(F):
        w1_pend.append(pltpu.async_copy(
            w1_hbm.at[idx_v.at[f]], w1_rows.at[f], w1sem))
    for h_ in w1_pend:
        h_.wait()

    for ch in range(BPW // L):
        acc = w1_rows[0, pl.ds(ch * L, L)]
        for f in range(1, F):
            acc = acc + w1_rows[f, pl.ds(ch * L, L)]
        fm1_stage[pl.ds(ch * L, L)] = acc
    pltpu.sync_copy(fm1_stage, fm1_hbm.at[pl.ds(base, BPW)])


_sc_b_call = pl.kernel(
    _sc_b_body,
    out_type=[jax.ShapeDtypeStruct((B,), jnp.float32)],
    mesh=plsc.VectorSubcoreMesh(core_axis_name="c", subcore_axis_name="s"),
    scratch_types=[
        pltpu.VMEM((F, BPW), jnp.int32),
        pltpu.VMEM((F, BPW), jnp.float32),
        pltpu.VMEM((BPW,), jnp.float32),
        pltpu.SemaphoreType.DMA,
    ],
    compiler_params=pltpu.CompilerParams(use_tc_tiling_on_sc=False),
)


def _tc_body(xbi_ref, fm1_ref, w0_ref, w1m_ref, b1_ref, w2m_ref, b2_ref,
             h_ref, out_ref):
    xbi = xbi_ref[:, :K]
    a1 = jnp.dot(xbi, w1m_ref[...], preferred_element_type=jnp.float32)
    a1 = jnp.maximum(a1 + b1_ref[...], 0.0)
    a2 = jnp.dot(a1, w2m_ref[...], preferred_element_type=jnp.float32)
    a2 = jnp.maximum(a2 + b2_ref[...], 0.0)
    out = jnp.dot(a2, h_ref[...], preferred_element_type=jnp.float32)
    out_ref[...] = out + fm1_ref[...] + w0_ref[0]


_TC_BLK = 512


def _tc_call(xbi, fm1_2d, w0_1, w1m, b1, w2m, b2, h):
    grid = (B // _TC_BLK,)
    return pl.pallas_call(
        _tc_body,
        grid=grid,
        in_specs=[
            pl.BlockSpec((_TC_BLK, 128), lambda i: (i, 0)),
            pl.BlockSpec((_TC_BLK, 1), lambda i: (i, 0)),
            pl.BlockSpec(memory_space=pltpu.SMEM),
            pl.BlockSpec((K, H1), lambda i: (0, 0)),
            pl.BlockSpec((H1,), lambda i: (0,)),
            pl.BlockSpec((H1, H2), lambda i: (0, 0)),
            pl.BlockSpec((H2,), lambda i: (0,)),
            pl.BlockSpec((H2, 1), lambda i: (0, 0)),
        ],
        out_specs=pl.BlockSpec((_TC_BLK, 1), lambda i: (i, 0)),
        out_shape=jax.ShapeDtypeStruct((B, 1), jnp.float32),
    )(xbi, fm1_2d, w0_1, w1m, b1, w2m, b2, h)


def kernel(x, emb_v, w1_table, w0, W1, b1, W2, b2, h):
    w1f = w1_table.reshape(V)
    (xbi,) = _sc_a_call(x, emb_v)
    (fm1,) = _sc_b_call(x, w1f)
    return _tc_call(xbi, fm1.reshape(B, 1), w0.reshape(1), W1, b1, W2, b2, h)
